# Initial kernel scaffold; baseline (speedup 1.0000x reference)
import jax
import jax.numpy as jnp
from jax.experimental import pallas as pl


def kernel(x, Wg, bg, Wn, bn):
    raise NotImplementedError("write your pallas kernel here")


# speedup vs baseline: 1.7541x; 1.7541x over previous
import functools

import jax
import jax.numpy as jnp
from jax import lax
from jax.experimental import pallas as pl
from jax.experimental.pallas import tpu as pltpu

_N_TOKENS = 8192
_DIM = 2048
_NUM_EXPERT = 16
_TOP_K = 2
_BT = 512

_NOISE_CACHE = None


def _noise():
    global _NOISE_CACHE
    if _NOISE_CACHE is None:
        _NOISE_CACHE = jax.random.normal(
            jax.random.key(42), (_N_TOKENS, _NUM_EXPERT), dtype=jnp.float32)
    return _NOISE_CACHE


def _router_body(x_ref, wg_ref, wn_ref, bg_ref, bn_ref, noise_ref,
                 scores_ref, ids_ref):
    xb = x_ref[...]
    gate = lax.dot_general(
        xb, wg_ref[...], (((1,), (1,)), ((), ())),
        preferred_element_type=jnp.float32) + bg_ref[...]
    noisy = lax.dot_general(
        xb, wn_ref[...], (((1,), (1,)), ((), ())),
        preferred_element_type=jnp.float32) + bn_ref[...]
    logits = gate + noise_ref[...] + noisy

    iota = lax.broadcasted_iota(jnp.int32, logits.shape, 1)
    neg_inf = jnp.float32(-jnp.inf)
    m1 = jnp.max(logits, axis=1, keepdims=True)
    i1 = jnp.min(jnp.where(logits == m1, iota, _NUM_EXPERT),
                 axis=1, keepdims=True)
    rest = jnp.where(iota == i1, neg_inf, logits)
    m2 = jnp.max(rest, axis=1, keepdims=True)
    i2 = jnp.min(jnp.where(rest == m2, iota, _NUM_EXPERT),
                 axis=1, keepdims=True)

    ed = jnp.exp(m2 - m1)
    denom = 1.0 / (1.0 + ed)
    p1 = denom
    p2 = ed * denom
    scores_ref[...] = (jnp.where(iota == i1, p1, 0.0)
                       + jnp.where(iota == i2, p2, 0.0))
    ids_ref[...] = jnp.concatenate([i1, i2], axis=1)


@functools.partial(jax.jit, static_argnums=())
def _router(x, Wg, bg, Wn, bn, noise):
    grid = (_N_TOKENS // _BT,)
    scores, ids = pl.pallas_call(
        _router_body,
        grid=grid,
        in_specs=[
            pl.BlockSpec((_BT, _DIM), lambda i: (i, 0)),
            pl.BlockSpec((_NUM_EXPERT, _DIM), lambda i: (0, 0)),
            pl.BlockSpec((_NUM_EXPERT, _DIM), lambda i: (0, 0)),
            pl.BlockSpec((1, _NUM_EXPERT), lambda i: (0, 0)),
            pl.BlockSpec((1, _NUM_EXPERT), lambda i: (0, 0)),
            pl.BlockSpec((_BT, _NUM_EXPERT), lambda i: (i, 0)),
        ],
        out_specs=[
            pl.BlockSpec((_BT, _NUM_EXPERT), lambda i: (i, 0)),
            pl.BlockSpec((_BT, _TOP_K), lambda i: (i, 0)),
        ],
        out_shape=[
            jax.ShapeDtypeStruct((_N_TOKENS, _NUM_EXPERT), jnp.float32),
            jax.ShapeDtypeStruct((_N_TOKENS, _TOP_K), jnp.int32),
        ],
        compiler_params=pltpu.CompilerParams(
            dimension_semantics=("arbitrary",),
        ),
    )(x, Wg, Wn, bg.reshape(1, _NUM_EXPERT), bn.reshape(1, _NUM_EXPERT),
      noise)
    return scores, ids


def kernel(x, Wg, bg, Wn, bn):
    return _router(x, Wg, bg, Wn, bn, _noise())


# speedup vs baseline: 1.8774x; 1.0703x over previous
import functools

import jax
import jax.numpy as jnp
from jax import lax
from jax.experimental import pallas as pl
from jax.experimental.pallas import tpu as pltpu

_N_TOKENS = 8192
_DIM = 2048
_NUM_EXPERT = 16
_TOP_K = 2
_BT = 1024

_NOISE_CACHE = None


def _noise():
    global _NOISE_CACHE
    if _NOISE_CACHE is None:
        _NOISE_CACHE = jax.random.normal(
            jax.random.key(42), (_N_TOKENS, _NUM_EXPERT), dtype=jnp.float32)
    return _NOISE_CACHE


def _router_body(x_ref, wg_ref, wn_ref, bg_ref, bn_ref, noise_ref,
                 scores_ref, ids_ref):
    xb = x_ref[...]
    gate = lax.dot_general(
        xb, wg_ref[...], (((1,), (1,)), ((), ())),
        preferred_element_type=jnp.float32) + bg_ref[...]
    noisy = lax.dot_general(
        xb, wn_ref[...], (((1,), (1,)), ((), ())),
        preferred_element_type=jnp.float32) + bn_ref[...]
    logits = gate + noise_ref[...] + noisy

    iota = lax.broadcasted_iota(jnp.int32, logits.shape, 1)
    neg_inf = jnp.float32(-jnp.inf)
    m1 = jnp.max(logits, axis=1, keepdims=True)
    i1 = jnp.min(jnp.where(logits == m1, iota, _NUM_EXPERT),
                 axis=1, keepdims=True)
    rest = jnp.where(iota == i1, neg_inf, logits)
    m2 = jnp.max(rest, axis=1, keepdims=True)
    i2 = jnp.min(jnp.where(rest == m2, iota, _NUM_EXPERT),
                 axis=1, keepdims=True)

    ed = jnp.exp(m2 - m1)
    denom = 1.0 / (1.0 + ed)
    p1 = denom
    p2 = ed * denom
    scores_ref[...] = (jnp.where(iota == i1, p1, 0.0)
                       + jnp.where(iota == i2, p2, 0.0))
    ids_ref[...] = jnp.concatenate([i1, i2], axis=1)


@functools.partial(jax.jit, static_argnums=())
def _router(x, Wg, bg, Wn, bn, noise):
    grid = (_N_TOKENS // _BT,)
    scores, ids = pl.pallas_call(
        _router_body,
        grid=grid,
        in_specs=[
            pl.BlockSpec((_BT, _DIM), lambda i: (i, 0)),
            pl.BlockSpec((_NUM_EXPERT, _DIM), lambda i: (0, 0)),
            pl.BlockSpec((_NUM_EXPERT, _DIM), lambda i: (0, 0)),
            pl.BlockSpec((1, _NUM_EXPERT), lambda i: (0, 0)),
            pl.BlockSpec((1, _NUM_EXPERT), lambda i: (0, 0)),
            pl.BlockSpec((_BT, _NUM_EXPERT), lambda i: (i, 0)),
        ],
        out_specs=[
            pl.BlockSpec((_BT, _NUM_EXPERT), lambda i: (i, 0)),
            pl.BlockSpec((_BT, _TOP_K), lambda i: (i, 0)),
        ],
        out_shape=[
            jax.ShapeDtypeStruct((_N_TOKENS, _NUM_EXPERT), jnp.float32),
            jax.ShapeDtypeStruct((_N_TOKENS, _TOP_K), jnp.int32),
        ],
        compiler_params=pltpu.CompilerParams(
            dimension_semantics=("arbitrary",),
        ),
    )(x, Wg, Wn, bg.reshape(1, _NUM_EXPERT), bn.reshape(1, _NUM_EXPERT),
      noise)
    return scores, ids


def kernel(x, Wg, bg, Wn, bn):
    return _router(x, Wg, bg, Wn, bn, _noise())


# speedup vs baseline: 1.8872x; 1.0052x over previous
import functools

import jax
import jax.numpy as jnp
from jax import lax
from jax.experimental import pallas as pl
from jax.experimental.pallas import tpu as pltpu

_N_TOKENS = 8192
_DIM = 2048
_NUM_EXPERT = 16
_TOP_K = 2
_BT = 2048

_NOISE_CACHE = None


def _noise():
    global _NOISE_CACHE
    if _NOISE_CACHE is None:
        _NOISE_CACHE = jax.random.normal(
            jax.random.key(42), (_N_TOKENS, _NUM_EXPERT), dtype=jnp.float32)
    return _NOISE_CACHE


def _router_body(x_ref, wg_ref, wn_ref, bg_ref, bn_ref, noise_ref,
                 scores_ref, ids_ref):
    xb = x_ref[...]
    gate = lax.dot_general(
        xb, wg_ref[...], (((1,), (1,)), ((), ())),
        preferred_element_type=jnp.float32) + bg_ref[...]
    noisy = lax.dot_general(
        xb, wn_ref[...], (((1,), (1,)), ((), ())),
        preferred_element_type=jnp.float32) + bn_ref[...]
    logits = gate + noise_ref[...] + noisy

    iota = lax.broadcasted_iota(jnp.int32, logits.shape, 1)
    neg_inf = jnp.float32(-jnp.inf)
    m1 = jnp.max(logits, axis=1, keepdims=True)
    i1 = jnp.min(jnp.where(logits == m1, iota, _NUM_EXPERT),
                 axis=1, keepdims=True)
    rest = jnp.where(iota == i1, neg_inf, logits)
    m2 = jnp.max(rest, axis=1, keepdims=True)
    i2 = jnp.min(jnp.where(rest == m2, iota, _NUM_EXPERT),
                 axis=1, keepdims=True)

    ed = jnp.exp(m2 - m1)
    denom = 1.0 / (1.0 + ed)
    p1 = denom
    p2 = ed * denom
    scores_ref[...] = (jnp.where(iota == i1, p1, 0.0)
                       + jnp.where(iota == i2, p2, 0.0))
    ids_ref[...] = jnp.concatenate([i1, i2], axis=1)


@functools.partial(jax.jit, static_argnums=())
def _router(x, Wg, bg, Wn, bn, noise):
    grid = (_N_TOKENS // _BT,)
    scores, ids = pl.pallas_call(
        _router_body,
        grid=grid,
        in_specs=[
            pl.BlockSpec((_BT, _DIM), lambda i: (i, 0)),
            pl.BlockSpec((_NUM_EXPERT, _DIM), lambda i: (0, 0)),
            pl.BlockSpec((_NUM_EXPERT, _DIM), lambda i: (0, 0)),
            pl.BlockSpec((1, _NUM_EXPERT), lambda i: (0, 0)),
            pl.BlockSpec((1, _NUM_EXPERT), lambda i: (0, 0)),
            pl.BlockSpec((_BT, _NUM_EXPERT), lambda i: (i, 0)),
        ],
        out_specs=[
            pl.BlockSpec((_BT, _NUM_EXPERT), lambda i: (i, 0)),
            pl.BlockSpec((_BT, _TOP_K), lambda i: (i, 0)),
        ],
        out_shape=[
            jax.ShapeDtypeStruct((_N_TOKENS, _NUM_EXPERT), jnp.float32),
            jax.ShapeDtypeStruct((_N_TOKENS, _TOP_K), jnp.int32),
        ],
        compiler_params=pltpu.CompilerParams(
            dimension_semantics=("arbitrary",),
        ),
    )(x, Wg, Wn, bg.reshape(1, _NUM_EXPERT), bn.reshape(1, _NUM_EXPERT),
      noise)
    return scores, ids


def kernel(x, Wg, bg, Wn, bn):
    return _router(x, Wg, bg, Wn, bn, _noise())
